# baseline (speedup 1.0000x reference)
import jax
import jax.numpy as jnp
from jax.experimental import pallas as pl
from jax.experimental.pallas import tpu as pltpu

B = 32
L = 2048
D_IN = 128
EMBED_DIM = 128
BLK = 32768


def _fused_kernel(x_ref, w_ref, b_ref, pos_ref, out_ref):
    acc = jnp.dot(x_ref[...], w_ref[...], preferred_element_type=jnp.float32)
    m = BLK // L
    acc = acc.reshape(m, L, EMBED_DIM) + pos_ref[...][None, :, :] + b_ref[...]
    out_ref[...] = acc.reshape(BLK, EMBED_DIM)


def kernel(x, W, b, pos_table):
    x2 = x.reshape(B * L, D_IN)
    b2 = b.reshape(1, EMBED_DIM)
    assert (B * L) % BLK == 0 and BLK % L == 0
    grid = (B * L) // BLK
    out = pl.pallas_call(
        _fused_kernel,
        grid=(grid,),
        in_specs=[
            pl.BlockSpec((BLK, D_IN), lambda i: (i, 0)),
            pl.BlockSpec((D_IN, EMBED_DIM), lambda i: (0, 0)),
            pl.BlockSpec((1, EMBED_DIM), lambda i: (0, 0)),
            pl.BlockSpec((L, EMBED_DIM), lambda i: (0, 0)),
        ],
        out_specs=pl.BlockSpec((BLK, EMBED_DIM), lambda i: (i, 0),
                               pipeline_mode=pl.Buffered(buffer_count=1)),
        out_shape=jax.ShapeDtypeStruct((B * L, EMBED_DIM), jnp.float32),
        compiler_params=pltpu.CompilerParams(
            dimension_semantics=("parallel",),
        ),
    )(x2, W, b2, pos_table)
    return out.reshape(B, L, EMBED_DIM)
